# baseline (speedup 1.0000x reference)
import jax
import jax.numpy as jnp
from jax.experimental import pallas as pl
from jax.experimental.pallas import tpu as pltpu

B, N, F, S, C, NOUT = 32, 32, 32, 4, 32, 16
K5 = (S + 1) * F * C
W5 = (S + 1) * C
AGG = S * N + N


def _iota2(shape, dim):
    return jax.lax.broadcasted_iota(jnp.int32, shape, dim)


def _net_kernel(x2_ref, a2_ref, e2_ref, w1_ref, wb1_ref, root1_ref, b1_ref,
                w2_ref, wb2_ref, root2_ref, b2_ref, dw_ref, db_ref,
                out_ref, ae_s, tu_s, r_s, h_s):
    f32 = jnp.float32
    feats = x2_ref[:, :F]

    fm_row = _iota2((F, K5), 0)
    fm_col = _iota2((F, K5), 1)
    fold_mask = ((fm_col % (F * C)) // C == fm_row).astype(f32)
    fc_row = _iota2((K5, W5), 0)
    fc_col = _iota2((K5, W5), 1)
    fold_col = ((fc_row // (F * C)) * C + (fc_row % C) == fc_col).astype(f32)
    p_row = _iota2((N * S, S * N), 0)
    p_col = _iota2((N * S, S * N), 1)
    perm = ((p_row % S) * N + p_row // S == p_col).astype(f32)
    x_row = _iota2((N, AGG), 0)
    x_col = _iota2((N, AGG), 1)
    expand = (x_col % N == x_row).astype(f32)
    s_row = _iota2((B, B * N), 0)
    s_col = _iota2((B, B * N), 1)
    poolsel = (s_col // N == s_row).astype(f32)

    et = jnp.dot(e2_ref[:], perm, preferred_element_type=f32)
    a5 = jnp.dot(a2_ref[:], expand, preferred_element_type=f32)
    ae_s[:] = a5 * jnp.concatenate(
        [et, jnp.ones((B * N, N), f32)], axis=1)

    def fold_weights(w_ref, wb_ref):
        srcs = [w_ref[s:s + 1, :] for s in range(S)] + [wb_ref[:]]
        stack = jnp.concatenate(srcs, axis=1)
        rep = jnp.broadcast_to(stack, (F, K5))
        return jnp.dot(rep * fold_mask, fold_col,
                       preferred_element_type=f32)

    def node_stage(src, w_ref, wb_ref, root_ref, b_ref):
        mcat = fold_weights(w_ref, wb_ref)
        tu_s[:] = jnp.dot(src, mcat, preferred_element_type=f32)
        r_s[:] = (jnp.dot(src, root_ref[:], preferred_element_type=f32)
                  + jnp.reshape(b_ref[:], (1, C)))

    def conv_rows(b):
        ae = ae_s[b * N:(b + 1) * N, :]
        tb = tu_s[b * N:(b + 1) * N, :]
        tu = jnp.concatenate(
            [tb[:, 0:C], tb[:, C:2 * C], tb[:, 2 * C:3 * C], tb[:, 3 * C:4 * C],
             tb[:, 4 * C:5 * C]], axis=0)
        rb = r_s[b * N:(b + 1) * N, :]
        return jnp.maximum(jnp.dot(ae, tu, preferred_element_type=f32) + rb, 0.0)

    node_stage(feats, w1_ref, wb1_ref, root1_ref, b1_ref)
    for b in range(B):
        h_s[b * N:(b + 1) * N, :] = conv_rows(b)

    node_stage(h_s[:], w2_ref, wb2_ref, root2_ref, b2_ref)
    for b in range(B):
        h_s[b * N:(b + 1) * N, :] = conv_rows(b)

    mcol = (x2_ref[:, F:F + 1] != 0.0).astype(f32)
    pooled = jnp.dot(poolsel, h_s[:] * mcol,
                     preferred_element_type=f32)
    out_ref[:] = (jnp.dot(pooled, dw_ref[:], preferred_element_type=f32)
                  + jnp.reshape(db_ref[:], (1, NOUT)))


def kernel(x, a, e, kn_w1, kn_b1, root1, bias1, kn_w2, kn_b2, root2, bias2,
           dense_w, dense_b):
    f32 = jnp.float32
    x2 = x.reshape(B * N, F + 1)
    a2 = a.reshape(B * N, N)
    e2 = e.reshape(B * N, N * S)
    wb1 = kn_b1.reshape(1, F * C)
    wb2 = kn_b2.reshape(1, C * C)

    return pl.pallas_call(
        _net_kernel,
        out_shape=jax.ShapeDtypeStruct((B, NOUT), f32),
        scratch_shapes=[
            pltpu.VMEM((B * N, AGG), f32),
            pltpu.VMEM((B * N, W5), f32),
            pltpu.VMEM((B * N, C), f32),
            pltpu.VMEM((B * N, C), f32),
        ],
    )(x2, a2, e2, kn_w1, wb1, root1, bias1,
      kn_w2, wb2, root2, bias2, dense_w, dense_b)


# speedup vs baseline: 1.0202x; 1.0202x over previous
import jax
import jax.numpy as jnp
from jax.experimental import pallas as pl
from jax.experimental.pallas import tpu as pltpu

B, N, F, S, C, NOUT = 32, 32, 32, 4, 32, 16


def _fold_machinery():
    f32 = jnp.float32
    row = jax.lax.broadcasted_iota(jnp.int32, (F, F * C), 0)
    col = jax.lax.broadcasted_iota(jnp.int32, (F, F * C), 1)
    blk_mask = (col // C == row).astype(f32)
    mrow = jax.lax.broadcasted_iota(jnp.int32, (F * C, C), 0)
    mcol = jax.lax.broadcasted_iota(jnp.int32, (F * C, C), 1)
    collapse = (mrow % C == mcol).astype(f32)
    return blk_mask, collapse


def _fold_row(w_row, blk_mask, collapse):
    rep = jnp.broadcast_to(w_row, (F, F * C))
    return jnp.dot(rep * blk_mask, collapse, preferred_element_type=jnp.float32)


def _net_kernel(x2_ref, a2_ref, e2_ref, w1_ref, wb1_ref, root1_ref, b1_ref,
                w2_ref, wb2_ref, root2_ref, b2_ref, dw_ref, db_ref,
                out_ref, ae_s, tu_s, r_s, h_s):
    f32 = jnp.float32
    bf16 = jnp.bfloat16
    blk_mask, collapse = _fold_machinery()
    feats = x2_ref[:, :F]

    prow = jax.lax.broadcasted_iota(jnp.int32, (N * S, S * N), 0)
    pcol = jax.lax.broadcasted_iota(jnp.int32, (N * S, S * N), 1)
    perm = ((prow % S) * N + prow // S == pcol).astype(f32)
    et = jnp.dot(e2_ref[:], perm, preferred_element_type=f32)

    a2 = a2_ref[:]
    a4 = jnp.concatenate([a2, a2, a2, a2], axis=1)
    ae_s[:] = jnp.concatenate([a4 * et, a2], axis=1).astype(bf16)

    def node_stage(src, w_ref, wb_ref, root_ref, b_ref):
        src_bf = src.astype(bf16)
        blocks = [jnp.dot(src_bf,
                          _fold_row(w_ref[s:s + 1, :], blk_mask,
                                    collapse).astype(bf16),
                          preferred_element_type=f32) for s in range(S)]
        blocks.append(jnp.dot(src_bf,
                              _fold_row(wb_ref[:], blk_mask,
                                        collapse).astype(bf16),
                              preferred_element_type=f32))
        tu_s[:] = jnp.concatenate(blocks, axis=1).astype(bf16)
        r_s[:] = (jnp.dot(src_bf, root_ref[:].astype(bf16),
                          preferred_element_type=f32)
                  + jnp.reshape(b_ref[:], (1, C)))

    def conv_rows(b):
        ae = ae_s[b * N:(b + 1) * N, :]
        tb = tu_s[b * N:(b + 1) * N, :]
        tu = jnp.concatenate(
            [tb[:, 0:C], tb[:, C:2 * C], tb[:, 2 * C:3 * C], tb[:, 3 * C:4 * C],
             tb[:, 4 * C:5 * C]], axis=0)
        rb = r_s[b * N:(b + 1) * N, :]
        return jnp.maximum(jnp.dot(ae, tu, preferred_element_type=f32) + rb, 0.0)

    node_stage(feats, w1_ref, wb1_ref, root1_ref, b1_ref)
    for b in range(B):
        h_s[b * N:(b + 1) * N, :] = conv_rows(b)

    node_stage(h_s[:], w2_ref, wb2_ref, root2_ref, b2_ref)

    mcol = (x2_ref[:, F:F + 1] != 0.0).astype(f32)
    rows = []
    for b in range(B):
        h2 = conv_rows(b)
        mb = mcol[b * N:(b + 1) * N, :]
        rows.append(jnp.sum(h2 * mb, axis=0, keepdims=True))
    pooled = jnp.concatenate(rows, axis=0)
    out_ref[:] = (jnp.dot(pooled.astype(bf16), dw_ref[:].astype(bf16),
                          preferred_element_type=f32)
                  + jnp.reshape(db_ref[:], (1, NOUT)))


def kernel(x, a, e, kn_w1, kn_b1, root1, bias1, kn_w2, kn_b2, root2, bias2,
           dense_w, dense_b):
    f32 = jnp.float32
    x2 = x.reshape(B * N, F + 1)
    a2 = a.reshape(B * N, N)
    e2 = e.reshape(B * N, N * S)
    wb1 = kn_b1.reshape(1, F * C)
    wb2 = kn_b2.reshape(1, C * C)

    return pl.pallas_call(
        _net_kernel,
        out_shape=jax.ShapeDtypeStruct((B, NOUT), f32),
        scratch_shapes=[
            pltpu.VMEM((B * N, S * N + N), jnp.bfloat16),
            pltpu.VMEM((B * N, (S + 1) * C), jnp.bfloat16),
            pltpu.VMEM((B * N, C), f32),
            pltpu.VMEM((B * N, C), f32),
        ],
    )(x2, a2, e2, kn_w1, wb1, root1, bias1,
      kn_w2, wb2, root2, bias2, dense_w, dense_b)
